# baseline (speedup 1.0000x reference)
import jax
import jax.numpy as jnp
from jax.experimental import pallas as pl
from jax.experimental.pallas import tpu as pltpu

_BMH = 200


def _gcn_kernel(features_ref, w_ref, a0_ref, a1_ref, out_ref, support_ref):
    @pl.when(pl.program_id(0) == 0)
    def _():
        support_ref[...] = jnp.dot(
            features_ref[...], w_ref[...], preferred_element_type=jnp.float32
        )

    s = support_ref[...]
    out_ref[0] = jnp.tanh(jnp.dot(a0_ref[0], s, preferred_element_type=jnp.float32))
    out_ref[1] = jnp.tanh(jnp.dot(a1_ref[0], s, preferred_element_type=jnp.float32))


def kernel(features, A, W):
    n, d_in = features.shape
    d_out = W.shape[1]
    half = n // 2
    a3 = A.reshape(2, half, n)
    out = pl.pallas_call(
        _gcn_kernel,
        grid=(half // _BMH,),
        in_specs=[
            pl.BlockSpec((n, d_in), lambda i: (0, 0)),
            pl.BlockSpec((d_in, d_out), lambda i: (0, 0)),
            pl.BlockSpec((1, _BMH, n), lambda i: (0, i, 0)),
            pl.BlockSpec((1, _BMH, n), lambda i: (1, i, 0)),
        ],
        out_specs=pl.BlockSpec((2, _BMH, d_out), lambda i: (0, i, 0)),
        out_shape=jax.ShapeDtypeStruct((2, half, d_out), jnp.float32),
        scratch_shapes=[pltpu.VMEM((n, d_out), jnp.float32)],
    )(features, W, a3, a3)
    return out.reshape(n, d_out)


# speedup vs baseline: 1.0178x; 1.0178x over previous
import jax
import jax.numpy as jnp
from jax.experimental import pallas as pl
from jax.experimental.pallas import tpu as pltpu

_BM = 400


def _gcn_kernel(features_ref, w_ref, a_ref, out_ref, support_ref):
    @pl.when(pl.program_id(0) == 0)
    def _():
        support_ref[...] = jnp.dot(
            features_ref[...], w_ref[...], preferred_element_type=jnp.float32
        )

    out_ref[...] = jnp.tanh(
        jnp.dot(a_ref[...], support_ref[...], preferred_element_type=jnp.float32)
    )


def kernel(features, A, W):
    n, d_in = features.shape
    d_out = W.shape[1]
    return pl.pallas_call(
        _gcn_kernel,
        grid=(n // _BM,),
        in_specs=[
            pl.BlockSpec((n, d_in), lambda i: (0, 0)),
            pl.BlockSpec((d_in, d_out), lambda i: (0, 0)),
            pl.BlockSpec((_BM, n), lambda i: (i, 0)),
        ],
        out_specs=pl.BlockSpec((_BM, d_out), lambda i: (i, 0)),
        out_shape=jax.ShapeDtypeStruct((n, d_out), jnp.float32),
        scratch_shapes=[pltpu.VMEM((n, d_out), jnp.float32)],
    )(features, W, A)
